# baseline (speedup 1.0000x reference)
import jax
import jax.numpy as jnp
from jax.experimental import pallas as pl
from jax.experimental.pallas import tpu as pltpu
from jax.experimental.pallas import tpu_sc as plsc

B, S, D = 1, 2048, 1024
H, DH, DHP = 16, 64, 128
E, F = 8, 4096
T = B * S
CAP = (2 * T) // E
ECAP = E * CAP
BM = 256
NB = T // BM
NEG = -1e9

f32 = jnp.float32
bf16 = jnp.bfloat16
i32 = jnp.int32


def _ln(xb, s_row, b_row):
    mu = jnp.mean(xb, axis=-1, keepdims=True)
    d = xb - mu
    var = jnp.mean(d * d, axis=-1, keepdims=True)
    return d / jnp.sqrt(var + 1e-5) * s_row + b_row


def _qkv_kernel(x_ref, s_ref, b_ref, wq_ref, wk_ref, wv_ref,
                q_ref, k_ref, v_ref):
    h1 = _ln(x_ref[...], s_ref[...], b_ref[...])
    q_ref[...] = jnp.dot(h1, wq_ref[...], preferred_element_type=f32).astype(bf16)
    k_ref[...] = jnp.dot(h1, wk_ref[...], preferred_element_type=f32).astype(bf16)
    v_ref[...] = jnp.dot(h1, wv_ref[...], preferred_element_type=f32).astype(bf16)


def _qkv(x2, s1, b1, wq, wk, wv):
    row = pl.BlockSpec((BM, D), lambda i: (i, 0))
    full = pl.BlockSpec((1, D), lambda i: (0, 0))
    w = pl.BlockSpec((D, D), lambda i: (0, 0))
    out = jax.ShapeDtypeStruct((T, D), bf16)
    return pl.pallas_call(
        _qkv_kernel,
        grid=(NB,),
        in_specs=[row, full, full, w, w, w],
        out_specs=[row, row, row],
        out_shape=[out, out, out],
    )(x2, s1, b1, wq, wk, wv)


def _attn_band_kernel(r0b, nc, q_ref, k_ref, v_ref, o_ref, mask_scr):
    i = pl.program_id(0)
    rio = jax.lax.broadcasted_iota(i32, (BM, nc), 0) + (r0b + i) * BM
    cio = jax.lax.broadcasted_iota(i32, (BM, nc), 1)
    mask_scr[...] = jnp.where(rio >= cio, 0.0, NEG)
    for h in range(H):
        sl = slice(h * DH, (h + 1) * DH)
        q = q_ref[:, sl]
        k = k_ref[:, sl]
        s = jax.lax.dot_general(
            q, k, (((1,), (1,)), ((), ())), preferred_element_type=f32)
        s = s * 0.125 + mask_scr[...]
        m = jnp.max(s, axis=1, keepdims=True)
        p = jnp.exp(s - m)
        rl = 1.0 / jnp.sum(p, axis=1, keepdims=True)
        attn = (p * rl).astype(bf16)
        o_ref[:, sl] = jnp.dot(attn, v_ref[:, sl],
                               preferred_element_type=f32).astype(bf16)


def _attention(q, k, v):
    import functools as _ft
    bands = []
    NBAND = 4
    rows = T // NBAND
    for bi in range(NBAND):
        r0 = bi * rows
        nc = r0 + rows
        band = pl.pallas_call(
            _ft.partial(_attn_band_kernel, r0 // BM, nc),
            grid=(rows // BM,),
            in_specs=[pl.BlockSpec((BM, D), lambda i, r0b=r0 // BM: (i + r0b, 0)),
                      pl.BlockSpec((nc, D), lambda i: (0, 0)),
                      pl.BlockSpec((nc, D), lambda i: (0, 0))],
            out_specs=pl.BlockSpec((BM, D), lambda i: (i, 0)),
            out_shape=jax.ShapeDtypeStruct((rows, D), bf16),
            scratch_shapes=[pltpu.VMEM((BM, nc), f32)],
        )(q, k, v)
        bands.append(band)
    return jnp.concatenate(bands, axis=0)


def _route_kernel(x_ref, o_ref, wo_ref, s_ref, b_ref, wr_ref,
                  x1_ref, h2_ref, slot_ref, ge_ref, cnt_ref):
    i = pl.program_id(0)
    x1 = x_ref[...] + jnp.dot(o_ref[...].astype(f32), wo_ref[...],
                              preferred_element_type=f32)
    x1_ref[...] = x1
    h2 = _ln(x1, s_ref[...], b_ref[...])
    h2_ref[...] = h2.astype(bf16)
    logits = jnp.dot(h2, wr_ref[...], preferred_element_type=f32)
    colio = jax.lax.broadcasted_iota(i32, (BM, 128), 1)
    logits = jnp.where(colio < E, logits, NEG)
    m = jnp.max(logits, axis=1, keepdims=True)
    p = jnp.exp(logits - m)
    z = jnp.sum(p, axis=1, keepdims=True)
    eidx = jnp.min(jnp.where(logits == m, colio, 127), axis=1, keepdims=True)
    onehot = (colio == eidx).astype(f32)
    gate = jnp.sum((p / z) * onehot, axis=1, keepdims=True)

    @pl.when(i == 0)
    def _():
        cnt_ref[...] = jnp.zeros((8, 128), f32)

    carry = cnt_ref[0:1, :]
    rio = jax.lax.broadcasted_iota(i32, (BM, BM), 0)
    cio = jax.lax.broadcasted_iota(i32, (BM, BM), 1)
    tril = (rio >= cio).astype(f32)
    cum = jnp.dot(tril, onehot, preferred_element_type=f32) + carry
    pos = jnp.sum(cum * onehot, axis=1, keepdims=True) - 1.0
    cnt_ref[0:1, :] = carry + jnp.sum(onehot, axis=0, keepdims=True)
    keep = pos < CAP
    slot = jnp.where(keep, eidx * CAP + pos.astype(i32), ECAP)
    slot_ref[...] = jnp.broadcast_to(slot, (BM, 128))
    ge_ref[...] = jnp.broadcast_to(jnp.where(keep, gate, 0.0), (BM, 128))


def _route(x2, o2, wo, s2, b2, wr_pad):
    rowf = pl.BlockSpec((BM, D), lambda i: (i, 0))
    full = pl.BlockSpec((1, D), lambda i: (0, 0))
    return pl.pallas_call(
        _route_kernel,
        grid=(NB,),
        in_specs=[rowf, rowf, pl.BlockSpec((D, D), lambda i: (0, 0)),
                  full, full, pl.BlockSpec((D, 128), lambda i: (0, 0))],
        out_specs=[rowf, rowf,
                   pl.BlockSpec((BM, 128), lambda i: (i, 0)),
                   pl.BlockSpec((BM, 128), lambda i: (i, 0))],
        out_shape=[jax.ShapeDtypeStruct((T, D), f32),
                   jax.ShapeDtypeStruct((T, D), bf16),
                   jax.ShapeDtypeStruct((T, 128), i32),
                   jax.ShapeDtypeStruct((T, 128), f32)],
        scratch_shapes=[pltpu.VMEM((8, 128), f32)],
    )(x2, o2, wo, s2, b2, wr_pad)


NF = 2
BF = F // NF


def _ffn_kernel(sr_ref, h2_ref, w1_ref, w2_ref, eout_ref, disp_scr):
    e = pl.program_id(0)
    fb = pl.program_id(1)

    @pl.when(fb == 0)
    def _():
        sr = sr_ref[0:1, :]
        rio = jax.lax.broadcasted_iota(i32, (CAP, T), 0) + e * CAP
        sel = (rio == sr).astype(bf16)
        disp_scr[...] = jnp.dot(sel, h2_ref[...],
                                preferred_element_type=f32).astype(bf16)

    h = jnp.dot(disp_scr[...].astype(f32), w1_ref[0], preferred_element_type=f32)
    h = jnp.maximum(h, 0.0)
    part = jnp.dot(h, w2_ref[0], preferred_element_type=f32)

    @pl.when(fb == 0)
    def _():
        eout_ref[0] = part

    @pl.when(fb > 0)
    def _():
        eout_ref[0] += part


def _ffn(slot_row, h2b, w1, w2):
    return pl.pallas_call(
        _ffn_kernel,
        grid=(E, NF),
        in_specs=[pl.BlockSpec((8, T), lambda e, fb: (0, 0)),
                  pl.BlockSpec((T, D), lambda e, fb: (0, 0)),
                  pl.BlockSpec((1, D, BF), lambda e, fb: (e, 0, fb)),
                  pl.BlockSpec((1, BF, D), lambda e, fb: (e, fb, 0))],
        out_specs=pl.BlockSpec((1, CAP, D), lambda e, fb: (e, 0, 0)),
        out_shape=jax.ShapeDtypeStruct((E, CAP, D), f32),
        scratch_shapes=[pltpu.VMEM((CAP, D), bf16)],
    )(slot_row, h2b, w1, w2)


GW = 128
DSPLIT = 4
DSUB = D // DSPLIT


def _gather_sc(eo_sub, idx_sub):
    mesh = plsc.VectorSubcoreMesh(core_axis_name="c", subcore_axis_name="s")
    n = T * DSPLIT

    @pl.kernel(out_type=jax.ShapeDtypeStruct((n, DSUB), f32), mesh=mesh)
    def kern(eo_hbm, i_hbm, o_hbm):
        def body(i_vmem, o_vmem):
            pltpu.sync_copy(eo_hbm.at[i_vmem.at[0]], o_vmem)

        pltpu.emit_pipeline(
            body,
            grid=(n // GW,),
            in_specs=[pl.BlockSpec((1, GW), index_map=lambda i: (0, i))],
            out_specs=[pl.BlockSpec((GW, DSUB), index_map=lambda i: (i, 0))],
            core_axis_name=("c", "s"),
            dimension_semantics=(pltpu.PARALLEL,),
        )(i_hbm, o_hbm)

    return kern(eo_sub, idx_sub)


def _resid_kernel(x1_ref, y_ref, ge_ref, out_ref):
    out_ref[...] = x1_ref[...] + ge_ref[:, 0:1] * y_ref[...]


def _resid(x1, y, ge_col):
    rowf = pl.BlockSpec((BM, D), lambda i: (i, 0))
    return pl.pallas_call(
        _resid_kernel,
        grid=(NB,),
        in_specs=[rowf, rowf, pl.BlockSpec((BM, 128), lambda i: (i, 0))],
        out_specs=rowf,
        out_shape=jax.ShapeDtypeStruct((T, D), f32),
    )(x1, y, ge_col)


def _combine_kernel(x1_ref, eo_ref, slot_ref, ge_ref, out_ref):
    sc = slot_ref[:, 0:1]
    cio = jax.lax.broadcasted_iota(i32, (BM, ECAP), 1)
    sel = (cio == sc).astype(f32)
    y = jnp.dot(sel, eo_ref[...], preferred_element_type=f32)
    out_ref[...] = x1_ref[...] + ge_ref[:, 0:1] * y


def _combine(x1, eo_flat, slot_col, ge_col):
    rowf = pl.BlockSpec((BM, D), lambda i: (i, 0))
    return pl.pallas_call(
        _combine_kernel,
        grid=(NB,),
        in_specs=[rowf,
                  pl.BlockSpec((ECAP, D), lambda i: (0, 0)),
                  pl.BlockSpec((BM, 128), lambda i: (i, 0)),
                  pl.BlockSpec((BM, 128), lambda i: (i, 0))],
        out_specs=rowf,
        out_shape=jax.ShapeDtypeStruct((T, D), f32),
    )(x1, eo_flat, slot_col, ge_col)


def kernel(x, Wq, Wk, Wv, Wo, ln1_s, ln1_b, ln2_s, ln2_b, Wr, W1, W2):
    x2 = x.reshape(T, D)
    s1 = ln1_s.reshape(1, D)
    b1 = ln1_b.reshape(1, D)
    s2 = ln2_s.reshape(1, D)
    b2 = ln2_b.reshape(1, D)
    wr_pad = jnp.pad(Wr, ((0, 0), (0, 128 - E)))

    q, k, v = _qkv(x2, s1, b1, Wq, Wk, Wv)
    o2 = _attention(q, k, v)
    x1, h2b, slot_col, ge_col = _route(x2, o2, Wo, s2, b2, wr_pad)

    slot_vec = slot_col[:, 0]
    slot_row = jnp.broadcast_to(slot_vec[None, :], (8, T))
    eout = _ffn(slot_row, h2b, W1, W2)
    slot_g = jnp.minimum(slot_vec, ECAP - 1)
    idx_sub = (slot_g[:, None] * DSPLIT
               + jnp.arange(DSPLIT, dtype=i32)[None, :]).reshape(1, T * DSPLIT)
    y = _gather_sc(eout.reshape(ECAP * DSPLIT, DSUB), idx_sub)
    out = _resid(x1, y.reshape(T, D), ge_col)
    return out.reshape(B, S, D)


# speedup vs baseline: 1.1278x; 1.1278x over previous
import jax
import jax.numpy as jnp
from jax.experimental import pallas as pl
from jax.experimental.pallas import tpu as pltpu

B, S, D = 1, 2048, 1024
H, DH, DHP = 16, 64, 128
E, F = 8, 4096
T = B * S
CAP = (2 * T) // E
ECAP = E * CAP
BM = 256
NB = T // BM
NEG = -1e9

f32 = jnp.float32
bf16 = jnp.bfloat16
i32 = jnp.int32


def _ln(xb, s_row, b_row):
    mu = jnp.mean(xb, axis=-1, keepdims=True)
    d = xb - mu
    var = jnp.mean(d * d, axis=-1, keepdims=True)
    return d / jnp.sqrt(var + 1e-5) * s_row + b_row


def _qkv_kernel(x_ref, s_ref, b_ref, wq_ref, wk_ref, wv_ref,
                q_ref, k_ref, v_ref):
    h1 = _ln(x_ref[...], s_ref[...], b_ref[...])
    q_ref[...] = jnp.dot(h1, wq_ref[...], preferred_element_type=f32).astype(bf16)
    k_ref[...] = jnp.dot(h1, wk_ref[...], preferred_element_type=f32).astype(bf16)
    v_ref[...] = jnp.dot(h1, wv_ref[...], preferred_element_type=f32).astype(bf16)


def _qkv(x2, s1, b1, wq, wk, wv):
    row = pl.BlockSpec((BM, D), lambda i: (i, 0))
    full = pl.BlockSpec((1, D), lambda i: (0, 0))
    w = pl.BlockSpec((D, D), lambda i: (0, 0))
    out = jax.ShapeDtypeStruct((T, D), bf16)
    return pl.pallas_call(
        _qkv_kernel,
        grid=(NB,),
        in_specs=[row, full, full, w, w, w],
        out_specs=[row, row, row],
        out_shape=[out, out, out],
    )(x2, s1, b1, wq, wk, wv)


def _attn_band_kernel(r0b, nc, q_ref, k_ref, v_ref, o_ref, mask_scr):
    i = pl.program_id(0)
    rio = jax.lax.broadcasted_iota(i32, (BM, nc), 0) + (r0b + i) * BM
    cio = jax.lax.broadcasted_iota(i32, (BM, nc), 1)
    mask_scr[...] = jnp.where(rio >= cio, 0.0, NEG)
    for h in range(H):
        sl = slice(h * DH, (h + 1) * DH)
        q = q_ref[:, sl]
        k = k_ref[:, sl]
        s = jax.lax.dot_general(
            q, k, (((1,), (1,)), ((), ())), preferred_element_type=f32)
        s = s * 0.125 + mask_scr[...]
        m = jnp.max(s, axis=1, keepdims=True)
        p = jnp.exp(s - m)
        rl = 1.0 / jnp.sum(p, axis=1, keepdims=True)
        attn = (p * rl).astype(bf16)
        o_ref[:, sl] = jnp.dot(attn, v_ref[:, sl],
                               preferred_element_type=f32).astype(bf16)


def _attention(q, k, v):
    import functools as _ft
    bands = []
    NBAND = 4
    rows = T // NBAND
    for bi in range(NBAND):
        r0 = bi * rows
        nc = r0 + rows
        band = pl.pallas_call(
            _ft.partial(_attn_band_kernel, r0 // BM, nc),
            grid=(rows // BM,),
            in_specs=[pl.BlockSpec((BM, D), lambda i, r0b=r0 // BM: (i + r0b, 0)),
                      pl.BlockSpec((nc, D), lambda i: (0, 0)),
                      pl.BlockSpec((nc, D), lambda i: (0, 0))],
            out_specs=pl.BlockSpec((BM, D), lambda i: (i, 0)),
            out_shape=jax.ShapeDtypeStruct((rows, D), bf16),
            scratch_shapes=[pltpu.VMEM((BM, nc), f32)],
        )(q, k, v)
        bands.append(band)
    return jnp.concatenate(bands, axis=0)


def _route_kernel(x_ref, o_ref, wo_ref, s_ref, b_ref, wr_ref,
                  x1_ref, h2_ref, slot_ref, ge_ref, cnt_ref):
    i = pl.program_id(0)
    x1 = x_ref[...] + jnp.dot(o_ref[...].astype(f32), wo_ref[...],
                              preferred_element_type=f32)
    x1_ref[...] = x1
    h2 = _ln(x1, s_ref[...], b_ref[...])
    h2_ref[...] = h2.astype(bf16)
    logits = jnp.dot(h2, wr_ref[...], preferred_element_type=f32)
    colio = jax.lax.broadcasted_iota(i32, (BM, 128), 1)
    logits = jnp.where(colio < E, logits, NEG)
    m = jnp.max(logits, axis=1, keepdims=True)
    p = jnp.exp(logits - m)
    z = jnp.sum(p, axis=1, keepdims=True)
    eidx = jnp.min(jnp.where(logits == m, colio, 127), axis=1, keepdims=True)
    onehot = (colio == eidx).astype(f32)
    gate = jnp.sum((p / z) * onehot, axis=1, keepdims=True)

    @pl.when(i == 0)
    def _():
        cnt_ref[...] = jnp.zeros((8, 128), f32)

    carry = cnt_ref[0:1, :]
    rio = jax.lax.broadcasted_iota(i32, (BM, BM), 0)
    cio = jax.lax.broadcasted_iota(i32, (BM, BM), 1)
    tril = (rio >= cio).astype(f32)
    cum = jnp.dot(tril, onehot, preferred_element_type=f32) + carry
    pos = jnp.sum(cum * onehot, axis=1, keepdims=True) - 1.0
    cnt_ref[0:1, :] = carry + jnp.sum(onehot, axis=0, keepdims=True)
    keep = pos < CAP
    slot = jnp.where(keep, eidx * CAP + pos.astype(i32), ECAP)
    slot_ref[...] = jnp.broadcast_to(slot, (BM, 128))
    ge_ref[...] = jnp.broadcast_to(jnp.where(keep, gate, 0.0), (BM, 128))


def _route(x2, o2, wo, s2, b2, wr_pad):
    rowf = pl.BlockSpec((BM, D), lambda i: (i, 0))
    full = pl.BlockSpec((1, D), lambda i: (0, 0))
    return pl.pallas_call(
        _route_kernel,
        grid=(NB,),
        in_specs=[rowf, rowf, pl.BlockSpec((D, D), lambda i: (0, 0)),
                  full, full, pl.BlockSpec((D, 128), lambda i: (0, 0))],
        out_specs=[rowf, rowf,
                   pl.BlockSpec((BM, 128), lambda i: (i, 0)),
                   pl.BlockSpec((BM, 128), lambda i: (i, 0))],
        out_shape=[jax.ShapeDtypeStruct((T, D), f32),
                   jax.ShapeDtypeStruct((T, D), bf16),
                   jax.ShapeDtypeStruct((T, 128), i32),
                   jax.ShapeDtypeStruct((T, 128), f32)],
        scratch_shapes=[pltpu.VMEM((8, 128), f32)],
    )(x2, o2, wo, s2, b2, wr_pad)


NF = 2
BF = F // NF


def _ffn_kernel(sr_ref, h2_ref, w1_ref, w2_ref, eout_ref, disp_scr, acc_scr):
    e = pl.program_id(0)
    fb = pl.program_id(1)

    @pl.when(fb == 0)
    def _():
        sr = sr_ref[0:1, :]
        rio = jax.lax.broadcasted_iota(i32, (CAP, T), 0) + e * CAP
        sel = (rio == sr).astype(bf16)
        disp_scr[...] = jnp.dot(sel, h2_ref[...],
                                preferred_element_type=f32).astype(bf16)

    h = jnp.dot(disp_scr[...].astype(f32), w1_ref[0], preferred_element_type=f32)
    h = jnp.maximum(h, 0.0)
    part = jnp.dot(h, w2_ref[0], preferred_element_type=f32)

    @pl.when(fb == 0)
    def _():
        acc_scr[...] = part

    @pl.when(fb > 0)
    def _():
        acc_scr[...] += part

    eout_ref[0] = acc_scr[...].astype(bf16)


def _ffn(slot_row, h2b, w1, w2):
    return pl.pallas_call(
        _ffn_kernel,
        grid=(E, NF),
        in_specs=[pl.BlockSpec((8, T), lambda e, fb: (0, 0)),
                  pl.BlockSpec((T, D), lambda e, fb: (0, 0)),
                  pl.BlockSpec((1, D, BF), lambda e, fb: (e, 0, fb)),
                  pl.BlockSpec((1, BF, D), lambda e, fb: (e, fb, 0))],
        out_specs=pl.BlockSpec((1, CAP, D), lambda e, fb: (e, 0, 0)),
        out_shape=jax.ShapeDtypeStruct((E, CAP, D), bf16),
        scratch_shapes=[pltpu.VMEM((CAP, D), bf16),
                        pltpu.VMEM((CAP, D), f32)],
    )(slot_row, h2b, w1, w2)


def _combine_kernel(x1_ref, eo_ref, slot_ref, ge_ref, out_ref):
    sc = slot_ref[:, 0:1]
    cio = jax.lax.broadcasted_iota(i32, (BM, ECAP), 1)
    sel = (cio == sc).astype(bf16)
    y = jnp.dot(sel, eo_ref[...], preferred_element_type=f32)
    out_ref[...] = x1_ref[...] + ge_ref[:, 0:1] * y


def _combine(x1, eo_flat, slot_col, ge_col):
    rowf = pl.BlockSpec((BM, D), lambda i: (i, 0))
    return pl.pallas_call(
        _combine_kernel,
        grid=(NB,),
        in_specs=[rowf,
                  pl.BlockSpec((ECAP, D), lambda i: (0, 0)),
                  pl.BlockSpec((BM, 128), lambda i: (i, 0)),
                  pl.BlockSpec((BM, 128), lambda i: (i, 0))],
        out_specs=rowf,
        out_shape=jax.ShapeDtypeStruct((T, D), f32),
    )(x1, eo_flat, slot_col, ge_col)


def kernel(x, Wq, Wk, Wv, Wo, ln1_s, ln1_b, ln2_s, ln2_b, Wr, W1, W2):
    x2 = x.reshape(T, D)
    s1 = ln1_s.reshape(1, D)
    b1 = ln1_b.reshape(1, D)
    s2 = ln2_s.reshape(1, D)
    b2 = ln2_b.reshape(1, D)
    wr_pad = jnp.pad(Wr, ((0, 0), (0, 128 - E)))

    q, k, v = _qkv(x2, s1, b1, Wq, Wk, Wv)
    o2 = _attention(q, k, v)
    x1, h2b, slot_col, ge_col = _route(x2, o2, Wo, s2, b2, wr_pad)

    slot_vec = slot_col[:, 0]
    slot_row = jnp.broadcast_to(slot_vec[None, :], (8, T))
    eout = _ffn(slot_row, h2b, W1, W2)
    out = _combine(x1, eout.reshape(ECAP, D), slot_col, ge_col)
    return out.reshape(B, S, D)
